# Initial kernel scaffold; baseline (speedup 1.0000x reference)
import jax
import jax.numpy as jnp
from jax.experimental import pallas as pl


def kernel(x, edge_index, W, a_src, a_dst, bias):
    raise NotImplementedError("write your pallas kernel here")


# speedup vs baseline: 30.2387x; 30.2387x over previous
import functools

import jax
import jax.numpy as jnp
from jax import lax
from jax.experimental import pallas as pl
from jax.experimental.pallas import tpu as pltpu
from jax.experimental.pallas import tpu_sc as plsc

N = 10000
E = 320000
D = 128
N_PAD = 10240
NW = 32
EDGES_PER_W = E // NW
CHUNK = 80
NCHUNK = EDGES_PER_W // CHUNK
IDX_BLK = 25
ROWS_PER_TILE = N_PAD // 16


def _proj_body(x_ref, w_ref, a2_ref, h_ref, al_ref, mx_ref):
    i = pl.program_id(0)
    h = jnp.dot(x_ref[...], w_ref[...], preferred_element_type=jnp.float32)
    h_ref[...] = h
    al = jnp.dot(h, a2_ref[...], preferred_element_type=jnp.float32)
    al_ref[...] = al
    bm = jnp.max(al, axis=0, keepdims=True)

    @pl.when(i == 0)
    def _():
        mx_ref[...] = bm

    @pl.when(i > 0)
    def _():
        mx_ref[...] = jnp.maximum(mx_ref[...], bm)


def _project(x, W, a2):
    grid = (10,)
    return pl.pallas_call(
        _proj_body,
        grid=grid,
        in_specs=[
            pl.BlockSpec((1000, 128), lambda i: (i, 0)),
            pl.BlockSpec((128, 128), lambda i: (0, 0)),
            pl.BlockSpec((128, 8), lambda i: (0, 0)),
        ],
        out_specs=[
            pl.BlockSpec((1000, 128), lambda i: (i, 0)),
            pl.BlockSpec((1000, 8), lambda i: (i, 0)),
            pl.BlockSpec((1, 8), lambda i: (0, 0)),
        ],
        out_shape=[
            jax.ShapeDtypeStruct((N, D), jnp.float32),
            jax.ShapeDtypeStruct((N, 8), jnp.float32),
            jax.ShapeDtypeStruct((1, 8), jnp.float32),
        ],
    )(x, W, a2)


def _edge_body(h_hbm, asrc_hbm, adst_hbm, sidx_hbm, didx_hbm, m_hbm,
               acc_out, dsum_out,
               asrc_v, adst_v, sidx_v, didx_v, rows_v, wbuf_v, zbuf_v,
               m_v, gsem,
               acc_sh, dsum_sh):
    c = lax.axis_index("c")
    s = lax.axis_index("s")
    wid = c * 16 + s

    zero16 = jnp.zeros((16,), jnp.float32)

    def zrow(r, carry):
        for g in range(8):
            zbuf_v[r, pl.ds(g * 16, 16)] = zero16
        return carry

    lax.fori_loop(0, 16, zrow, 0)

    def zcp(k, carry):
        pltpu.sync_copy(zbuf_v, acc_sh.at[pl.ds(s * ROWS_PER_TILE + k * 16, 16)])
        return carry

    lax.fori_loop(0, ROWS_PER_TILE // 16, zcp, 0)

    def zdcp(k, carry):
        pltpu.sync_copy(zbuf_v.at[0],
                        dsum_sh.at[pl.ds(s * ROWS_PER_TILE + k * 128, 128)])
        return carry

    lax.fori_loop(0, ROWS_PER_TILE // 128, zdcp, 0)

    pltpu.sync_copy(asrc_hbm, asrc_v)
    pltpu.sync_copy(adst_hbm, adst_v)
    pltpu.sync_copy(m_hbm, m_v)

    mvec = m_v[...]

    plsc.subcore_barrier()

    def chunk(j, carry):
        p = j // IDX_BLK
        jj = j % IDX_BLK

        @pl.when(jj == 0)
        def _():
            pltpu.sync_copy(sidx_hbm.at[wid, p], sidx_v)
            pltpu.sync_copy(didx_hbm.at[wid, p], didx_v)

        gdma = pltpu.async_copy(h_hbm.at[sidx_v.at[jj]], rows_v, gsem)
        lane_iota = lax.iota(jnp.int32, 16)
        ws = []
        for g in range(5):
            vs = sidx_v[jj, pl.ds(g * 16, 16)]
            vd = didx_v[jj, pl.ds(g * 16, 16)]
            a1 = plsc.load_gather(asrc_v, [vs])
            a2 = plsc.load_gather(adst_v, [vd])
            e = a1 + a2
            e = jnp.where(e >= 0.0, e, 0.2 * e)
            w = jnp.exp(e - mvec)
            wbuf_v[pl.ds(g * 16, 16)] = w
            ws.append(w)
        gdma.wait()
        for g in range(5):
            for l in range(16):
                wl = jnp.sum(jnp.where(lane_iota == l, ws[g], 0.0))
                r = g * 16 + l
                for f in range(8):
                    sl = pl.ds(f * 16, 16)
                    rows_v[r, sl] = rows_v[r, sl] * wl
        pltpu.sync_copy(rows_v, acc_sh.at[didx_v.at[jj]], add=True)
        pltpu.sync_copy(wbuf_v, dsum_sh.at[didx_v.at[jj]], add=True)
        return carry

    lax.fori_loop(0, NCHUNK, chunk, 0)

    plsc.subcore_barrier()

    pltpu.sync_copy(acc_sh.at[pl.ds(s * ROWS_PER_TILE, ROWS_PER_TILE)],
                    acc_out.at[c, pl.ds(s * ROWS_PER_TILE, ROWS_PER_TILE)])
    pltpu.sync_copy(dsum_sh.at[pl.ds(s * ROWS_PER_TILE, ROWS_PER_TILE)],
                    dsum_out.at[c, pl.ds(s * ROWS_PER_TILE, ROWS_PER_TILE)])


def _edge_phase(h, asrc, adst, sidx, didx, m_arr):
    mesh = plsc.VectorSubcoreMesh(core_axis_name="c", subcore_axis_name="s",
                                  num_cores=2, num_subcores=16)
    k = pl.kernel(
        _edge_body,
        out_type=[
            jax.ShapeDtypeStruct((2, N_PAD, D), jnp.float32),
            jax.ShapeDtypeStruct((2, N_PAD), jnp.float32),
        ],
        mesh=mesh,
        compiler_params=pltpu.CompilerParams(needs_layout_passes=False),
        scratch_types=[
            pltpu.VMEM((N,), jnp.float32),
            pltpu.VMEM((N,), jnp.float32),
            pltpu.VMEM((IDX_BLK, CHUNK), jnp.int32),
            pltpu.VMEM((IDX_BLK, CHUNK), jnp.int32),
            pltpu.VMEM((CHUNK, D), jnp.float32),
            pltpu.VMEM((CHUNK,), jnp.float32),
            pltpu.VMEM((16, D), jnp.float32),
            pltpu.VMEM((16,), jnp.float32),
            pltpu.SemaphoreType.DMA,
            pltpu.VMEM_SHARED((N_PAD, D), jnp.float32),
            pltpu.VMEM_SHARED((N_PAD,), jnp.float32),
        ],
    )
    return k(h, asrc, adst, sidx, didx, m_arr)


def _final_body(a0_ref, a1_ref, d0_ref, d1_ref, b_ref, out_ref):
    t = a0_ref[...] + a1_ref[...]
    d = d0_ref[...] + d1_ref[...] + 1e-16
    r = t / d + b_ref[...]
    out_ref[...] = jnp.where(r > 0.0, r, jnp.exp(r) - 1.0)


def _finalize(a0, a1, d0, d1, bias2):
    grid = (10,)
    return pl.pallas_call(
        _final_body,
        grid=grid,
        in_specs=[
            pl.BlockSpec((1000, 128), lambda i: (i, 0)),
            pl.BlockSpec((1000, 128), lambda i: (i, 0)),
            pl.BlockSpec((1000, 1), lambda i: (i, 0)),
            pl.BlockSpec((1000, 1), lambda i: (i, 0)),
            pl.BlockSpec((1, 128), lambda i: (0, 0)),
        ],
        out_specs=pl.BlockSpec((1000, 128), lambda i: (i, 0)),
        out_shape=jax.ShapeDtypeStruct((N, D), jnp.float32),
    )(a0, a1, d0, d1, bias2)


@jax.jit
def kernel(x, edge_index, W, a_src, a_dst, bias):
    a2 = jnp.zeros((D, 8), jnp.float32).at[:, 0].set(a_src).at[:, 1].set(a_dst)
    h, al, mx = _project(x, W, a2)
    asrc = al[:, 0]
    adst = al[:, 1]
    m = mx[0, 0] + mx[0, 1]
    m = jnp.where(m >= 0.0, m, 0.2 * m)
    m_arr = jnp.full((16,), m, jnp.float32)

    sidx = edge_index[0].reshape(NW, NCHUNK // IDX_BLK, IDX_BLK, CHUNK)
    didx = edge_index[1].reshape(NW, NCHUNK // IDX_BLK, IDX_BLK, CHUNK)

    acc, dsum = _edge_phase(h, asrc, adst, sidx, didx, m_arr)

    out = _finalize(acc[0], acc[1], dsum[0][:, None], dsum[1][:, None],
                    bias[None, :])
    return out


# speedup vs baseline: 42.2432x; 1.3970x over previous
import functools

import jax
import jax.numpy as jnp
from jax import lax
from jax.experimental import pallas as pl
from jax.experimental.pallas import tpu as pltpu
from jax.experimental.pallas import tpu_sc as plsc

N = 10000
E = 320000
D = 128
N_PAD = 10240
NW = 32
EDGES_PER_W = E // NW
CHUNK = 80
NCHUNK = EDGES_PER_W // CHUNK
ROWS_PER_TILE = N_PAD // 16


def _proj_body(x_ref, w_ref, a2_ref, h_ref, al_ref, mx_ref):
    i = pl.program_id(0)
    h = jnp.dot(x_ref[...], w_ref[...], preferred_element_type=jnp.float32)
    h_ref[...] = h
    al = jnp.dot(h, a2_ref[...], preferred_element_type=jnp.float32)
    al_ref[...] = al
    bm = jnp.max(al, axis=0, keepdims=True)

    @pl.when(i == 0)
    def _():
        mx_ref[...] = bm

    @pl.when(i > 0)
    def _():
        mx_ref[...] = jnp.maximum(mx_ref[...], bm)


def _project(x, W, a2):
    grid = (10,)
    return pl.pallas_call(
        _proj_body,
        grid=grid,
        in_specs=[
            pl.BlockSpec((1000, 128), lambda i: (i, 0)),
            pl.BlockSpec((128, 128), lambda i: (0, 0)),
            pl.BlockSpec((128, 8), lambda i: (0, 0)),
        ],
        out_specs=[
            pl.BlockSpec((1000, 128), lambda i: (i, 0)),
            pl.BlockSpec((1000, 8), lambda i: (i, 0)),
            pl.BlockSpec((1, 8), lambda i: (0, 0)),
        ],
        out_shape=[
            jax.ShapeDtypeStruct((N, D), jnp.float32),
            jax.ShapeDtypeStruct((N, 8), jnp.float32),
            jax.ShapeDtypeStruct((1, 8), jnp.float32),
        ],
    )(x, W, a2)


def _edge_body(h_hbm, asrc_hbm, adst_hbm, sidx_hbm, didx_hbm, m_hbm,
               acc_out, dsum_out,
               sidx4, didx4, rows0, rows1, wb0, wb1, a1b0, a1b1,
               a2b0, a2b1, m_v,
               g0, g1, ga0, ga1, gb0, gb1, sa0, sa1, sd0, sd1,
               i0, i1, jj0, jj1,
               acc_sh, dsum_sh):
    c = lax.axis_index("c")
    s = lax.axis_index("s")
    wid = c * 16 + s

    rows = (rows0, rows1)
    wb = (wb0, wb1)
    a1b = (a1b0, a1b1)
    a2b = (a2b0, a2b1)
    gsem = (g0, g1)
    gasem = (ga0, ga1)
    gbsem = (gb0, gb1)
    sasem = (sa0, sa1)
    sdsem = (sd0, sd1)
    isem = (i0, i1)
    jsem = (jj0, jj1)

    zero16 = jnp.zeros((16,), jnp.float32)

    def zrow(r, carry):
        for g in range(8):
            rows0[r, pl.ds(g * 16, 16)] = zero16
        return carry

    lax.fori_loop(0, 16, zrow, 0)
    z16 = rows0.at[pl.ds(0, 16)]

    def zcp(k, carry):
        pltpu.sync_copy(z16, acc_sh.at[pl.ds(s * ROWS_PER_TILE + k * 16, 16)])
        return carry

    lax.fori_loop(0, ROWS_PER_TILE // 16, zcp, 0)

    def zdcp(k, carry):
        pltpu.sync_copy(rows0.at[0],
                        dsum_sh.at[pl.ds(s * ROWS_PER_TILE + k * 128, 128)])
        return carry

    lax.fori_loop(0, ROWS_PER_TILE // 128, zdcp, 0)

    pltpu.sync_copy(m_hbm, m_v)
    mvec = m_v[...]

    plsc.subcore_barrier()

    def stage_idx(j, b):
        pltpu.async_copy(sidx_hbm.at[wid, j], sidx4.at[j % 4], isem[b])
        pltpu.async_copy(didx_hbm.at[wid, j], didx4.at[j % 4], jsem[b])

    def wait_idx(j, b):
        pltpu.make_async_copy(sidx_hbm.at[wid, j], sidx4.at[j % 4],
                              isem[b]).wait()
        pltpu.make_async_copy(didx_hbm.at[wid, j], didx4.at[j % 4],
                              jsem[b]).wait()

    def issue_gathers(j, b):
        pltpu.async_copy(h_hbm.at[sidx4.at[j % 4]], rows[b], gsem[b])
        pltpu.async_copy(asrc_hbm.at[sidx4.at[j % 4]], a1b[b], gasem[b])
        pltpu.async_copy(adst_hbm.at[didx4.at[j % 4]], a2b[b], gbsem[b])

    def issue_scatters(j, b):
        pltpu.async_copy(rows[b], acc_sh.at[didx4.at[j % 4]], sasem[b],
                         add=True)
        pltpu.async_copy(wb[b], dsum_sh.at[didx4.at[j % 4]], sdsem[b],
                         add=True)

    def wait_scatters(j, b):
        pltpu.make_async_copy(rows[b], acc_sh.at[didx4.at[j % 4]],
                              sasem[b]).wait()
        pltpu.make_async_copy(wb[b], dsum_sh.at[didx4.at[j % 4]],
                              sdsem[b]).wait()

    def do_chunk(j, b, steady, guard_prev):
        pltpu.make_async_copy(asrc_hbm.at[sidx4.at[j % 4]], a1b[b],
                              gasem[b]).wait()
        pltpu.make_async_copy(adst_hbm.at[didx4.at[j % 4]], a2b[b],
                              gbsem[b]).wait()
        ws = []
        for g in range(5):
            sl = pl.ds(g * 16, 16)
            e = a1b[b][sl] + a2b[b][sl]
            e = jnp.where(e >= 0.0, e, 0.2 * e)
            w = jnp.exp(e - mvec)
            wb[b][sl] = w
            ws.append(w)
        pltpu.make_async_copy(h_hbm.at[sidx4.at[j % 4]], rows[b],
                              gsem[b]).wait()
        if guard_prev:
            @pl.when(j > 0)
            def _():
                wait_scatters(j - 1, 1 - b)
        else:
            wait_scatters(j - 1, 1 - b)
        if steady:
            @pl.when(j + 2 < NCHUNK)
            def _():
                stage_idx(j + 2, b)

            wait_idx(j + 1, 1 - b)
            issue_gathers(j + 1, 1 - b)
        for g in range(5):
            for l in range(16):
                wl = ws[g][l]
                r = g * 16 + l
                for f in range(8):
                    sl = pl.ds(f * 16, 16)
                    rows[b][r, sl] = rows[b][r, sl] * wl
        issue_scatters(j, b)

    pltpu.sync_copy(sidx_hbm.at[wid, 0], sidx4.at[0])
    pltpu.sync_copy(didx_hbm.at[wid, 0], didx4.at[0])
    stage_idx(1, 1)
    issue_gathers(0, 0)

    def pair(t, carry):
        do_chunk(2 * t, 0, steady=True, guard_prev=True)
        do_chunk(2 * t + 1, 1, steady=True, guard_prev=False)
        return carry

    lax.fori_loop(0, (NCHUNK - 1) // 2, pair, 0)

    jl = NCHUNK - 1
    do_chunk(jl, 0, steady=False, guard_prev=False)
    wait_scatters(jl, 0)

    plsc.subcore_barrier()

    pltpu.sync_copy(acc_sh.at[pl.ds(s * ROWS_PER_TILE, ROWS_PER_TILE)],
                    acc_out.at[c, pl.ds(s * ROWS_PER_TILE, ROWS_PER_TILE)])
    pltpu.sync_copy(dsum_sh.at[pl.ds(s * ROWS_PER_TILE, ROWS_PER_TILE)],
                    dsum_out.at[c, pl.ds(s * ROWS_PER_TILE, ROWS_PER_TILE)])


def _edge_phase(h, asrc, adst, sidx, didx, m_arr):
    mesh = plsc.VectorSubcoreMesh(core_axis_name="c", subcore_axis_name="s",
                                  num_cores=2, num_subcores=16)
    k = pl.kernel(
        _edge_body,
        out_type=[
            jax.ShapeDtypeStruct((2, N_PAD, D), jnp.float32),
            jax.ShapeDtypeStruct((2, N_PAD), jnp.float32),
        ],
        mesh=mesh,
        compiler_params=pltpu.CompilerParams(needs_layout_passes=False),
        scratch_types=[
            pltpu.VMEM((4, CHUNK), jnp.int32),
            pltpu.VMEM((4, CHUNK), jnp.int32),
            pltpu.VMEM((CHUNK, D), jnp.float32),
            pltpu.VMEM((CHUNK, D), jnp.float32),
            pltpu.VMEM((CHUNK,), jnp.float32),
            pltpu.VMEM((CHUNK,), jnp.float32),
            pltpu.VMEM((CHUNK,), jnp.float32),
            pltpu.VMEM((CHUNK,), jnp.float32),
            pltpu.VMEM((CHUNK,), jnp.float32),
            pltpu.VMEM((CHUNK,), jnp.float32),
            pltpu.VMEM((16,), jnp.float32),
            pltpu.SemaphoreType.DMA,
            pltpu.SemaphoreType.DMA,
            pltpu.SemaphoreType.DMA,
            pltpu.SemaphoreType.DMA,
            pltpu.SemaphoreType.DMA,
            pltpu.SemaphoreType.DMA,
            pltpu.SemaphoreType.DMA,
            pltpu.SemaphoreType.DMA,
            pltpu.SemaphoreType.DMA,
            pltpu.SemaphoreType.DMA,
            pltpu.SemaphoreType.DMA,
            pltpu.SemaphoreType.DMA,
            pltpu.SemaphoreType.DMA,
            pltpu.SemaphoreType.DMA,
            pltpu.VMEM_SHARED((N_PAD, D), jnp.float32),
            pltpu.VMEM_SHARED((N_PAD,), jnp.float32),
        ],
    )
    return k(h, asrc, adst, sidx, didx, m_arr)


def _final_body(a0_ref, a1_ref, d0_ref, d1_ref, b_ref, out_ref):
    t = a0_ref[...] + a1_ref[...]
    d = d0_ref[...] + d1_ref[...] + 1e-16
    r = t / d + b_ref[...]
    out_ref[...] = jnp.where(r > 0.0, r, jnp.exp(r) - 1.0)


def _finalize(a0, a1, d0, d1, bias2):
    grid = (10,)
    return pl.pallas_call(
        _final_body,
        grid=grid,
        in_specs=[
            pl.BlockSpec((1000, 128), lambda i: (i, 0)),
            pl.BlockSpec((1000, 128), lambda i: (i, 0)),
            pl.BlockSpec((1000, 1), lambda i: (i, 0)),
            pl.BlockSpec((1000, 1), lambda i: (i, 0)),
            pl.BlockSpec((1, 128), lambda i: (0, 0)),
        ],
        out_specs=pl.BlockSpec((1000, 128), lambda i: (i, 0)),
        out_shape=jax.ShapeDtypeStruct((N, D), jnp.float32),
    )(a0, a1, d0, d1, bias2)


@jax.jit
def kernel(x, edge_index, W, a_src, a_dst, bias):
    a2 = jnp.zeros((D, 8), jnp.float32).at[:, 0].set(a_src).at[:, 1].set(a_dst)
    h, al, mx = _project(x, W, a2)
    asrc = al[:, 0]
    adst = al[:, 1]
    m = mx[0, 0] + mx[0, 1]
    m = jnp.where(m >= 0.0, m, 0.2 * m)
    m_arr = jnp.full((16,), m, jnp.float32)

    sidx = edge_index[0].reshape(NW, NCHUNK, CHUNK)
    didx = edge_index[1].reshape(NW, NCHUNK, CHUNK)

    acc, dsum = _edge_phase(h, asrc, adst, sidx, didx, m_arr)

    out = _finalize(acc[0], acc[1], dsum[0][:, None], dsum[1][:, None],
                    bias[None, :])
    return out


# speedup vs baseline: 44.6112x; 1.0561x over previous
import functools

import jax
import jax.numpy as jnp
from jax import lax
from jax.experimental import pallas as pl
from jax.experimental.pallas import tpu as pltpu
from jax.experimental.pallas import tpu_sc as plsc

N = 10000
E = 320000
D = 128
N_PAD = 10240
NW = 32
EDGES_PER_W = E // NW
CHUNK = 80
NCHUNK = EDGES_PER_W // CHUNK
ROWS_PER_TILE = N_PAD // 16


def _proj_body(x_ref, w_ref, a2_ref, h_ref, al_ref, mx_ref):
    i = pl.program_id(0)
    h = jnp.dot(x_ref[...], w_ref[...], preferred_element_type=jnp.float32)
    h_ref[...] = h
    al = jnp.dot(h, a2_ref[...], preferred_element_type=jnp.float32)
    al_ref[...] = al
    bm = jnp.max(al, axis=0, keepdims=True)

    @pl.when(i == 0)
    def _():
        mx_ref[...] = bm

    @pl.when(i > 0)
    def _():
        mx_ref[...] = jnp.maximum(mx_ref[...], bm)


def _project(x, W, a2):
    grid = (10,)
    return pl.pallas_call(
        _proj_body,
        grid=grid,
        in_specs=[
            pl.BlockSpec((1000, 128), lambda i: (i, 0)),
            pl.BlockSpec((128, 128), lambda i: (0, 0)),
            pl.BlockSpec((128, 8), lambda i: (0, 0)),
        ],
        out_specs=[
            pl.BlockSpec((1000, 128), lambda i: (i, 0)),
            pl.BlockSpec((1000, 8), lambda i: (i, 0)),
            pl.BlockSpec((1, 8), lambda i: (0, 0)),
        ],
        out_shape=[
            jax.ShapeDtypeStruct((N, D), jnp.float32),
            jax.ShapeDtypeStruct((N, 8), jnp.float32),
            jax.ShapeDtypeStruct((1, 8), jnp.float32),
        ],
    )(x, W, a2)


def _edge_body(h_hbm, asrc_hbm, adst_hbm, sidx_hbm, m_hbm,
               acc_out, dsum_out,
               sidx4, didx4, rows0, rows1, wb0, wb1, a1b0, a1b1,
               a2b0, a2b1, m_v,
               g0, g1, ga0, ga1, gb0, gb1, sa0, sa1, sd0, sd1,
               i0, i1, jj0, jj1,
               acc_sh, dsum_sh):
    c = lax.axis_index("c")
    s = lax.axis_index("s")
    wid = c * 16 + s

    rows = (rows0, rows1)
    wb = (wb0, wb1)
    a1b = (a1b0, a1b1)
    a2b = (a2b0, a2b1)
    gsem = (g0, g1)
    gasem = (ga0, ga1)
    gbsem = (gb0, gb1)
    sasem = (sa0, sa1)
    sdsem = (sd0, sd1)
    isem = (i0, i1)
    jsem = (jj0, jj1)

    zero16 = jnp.zeros((16,), jnp.float32)

    def zrow(r, carry):
        for g in range(8):
            rows0[r, pl.ds(g * 16, 16)] = zero16
        return carry

    lax.fori_loop(0, 16, zrow, 0)
    z16 = rows0.at[pl.ds(0, 16)]

    def zcp(k, carry):
        pltpu.sync_copy(z16, acc_sh.at[pl.ds(s * ROWS_PER_TILE + k * 16, 16)])
        return carry

    lax.fori_loop(0, ROWS_PER_TILE // 16, zcp, 0)

    def zdcp(k, carry):
        pltpu.sync_copy(rows0.at[0],
                        dsum_sh.at[pl.ds(s * ROWS_PER_TILE + k * 128, 128)])
        return carry

    lax.fori_loop(0, ROWS_PER_TILE // 128, zdcp, 0)

    pltpu.sync_copy(m_hbm, m_v)
    mvec = m_v[...]

    plsc.subcore_barrier()

    def stage_idx(j, b):
        off = (wid * NCHUNK + j) * CHUNK
        pltpu.async_copy(sidx_hbm.at[pl.ds(off, CHUNK)], sidx4.at[j % 4],
                         isem[b])
        pltpu.async_copy(sidx_hbm.at[pl.ds(E + off, CHUNK)], didx4.at[j % 4],
                         jsem[b])

    def wait_idx(j, b):
        off = (wid * NCHUNK + j) * CHUNK
        pltpu.make_async_copy(sidx_hbm.at[pl.ds(off, CHUNK)],
                              sidx4.at[j % 4], isem[b]).wait()
        pltpu.make_async_copy(sidx_hbm.at[pl.ds(E + off, CHUNK)],
                              didx4.at[j % 4], jsem[b]).wait()

    def issue_gathers(j, b):
        pltpu.async_copy(h_hbm.at[sidx4.at[j % 4]], rows[b], gsem[b])
        pltpu.async_copy(asrc_hbm.at[sidx4.at[j % 4]], a1b[b], gasem[b])
        pltpu.async_copy(adst_hbm.at[didx4.at[j % 4]], a2b[b], gbsem[b])

    def issue_scatters(j, b):
        pltpu.async_copy(rows[b], acc_sh.at[didx4.at[j % 4]], sasem[b],
                         add=True)

    def wait_scatters(j, b):
        pltpu.make_async_copy(rows[b], acc_sh.at[didx4.at[j % 4]],
                              sasem[b]).wait()
        pltpu.make_async_copy(wb[b], dsum_sh.at[didx4.at[j % 4]],
                              sdsem[b]).wait()

    def do_chunk(j, b, steady, guard_prev):
        pltpu.make_async_copy(asrc_hbm.at[sidx4.at[j % 4]], a1b[b],
                              gasem[b]).wait()
        pltpu.make_async_copy(adst_hbm.at[didx4.at[j % 4]], a2b[b],
                              gbsem[b]).wait()
        ws = []
        for g in range(5):
            sl = pl.ds(g * 16, 16)
            e = a1b[b][sl] + a2b[b][sl]
            e = jnp.where(e >= 0.0, e, 0.2 * e)
            w = jnp.exp(e - mvec)
            wb[b][sl] = w
            ws.append(w)
        pltpu.async_copy(wb[b], dsum_sh.at[didx4.at[j % 4]], sdsem[b],
                         add=True)
        pltpu.make_async_copy(h_hbm.at[sidx4.at[j % 4]], rows[b],
                              gsem[b]).wait()
        if guard_prev:
            @pl.when(j > 0)
            def _():
                wait_scatters(j - 1, 1 - b)
        else:
            wait_scatters(j - 1, 1 - b)
        if steady:
            @pl.when(j + 2 < NCHUNK)
            def _():
                stage_idx(j + 2, b)

            wait_idx(j + 1, 1 - b)
            issue_gathers(j + 1, 1 - b)
        for g in range(5):
            for l in range(16):
                wl = ws[g][l]
                r = g * 16 + l
                for f in range(8):
                    sl = pl.ds(f * 16, 16)
                    rows[b][r, sl] = rows[b][r, sl] * wl
        issue_scatters(j, b)

    off0 = wid * NCHUNK * CHUNK
    pltpu.sync_copy(sidx_hbm.at[pl.ds(off0, CHUNK)], sidx4.at[0])
    pltpu.sync_copy(sidx_hbm.at[pl.ds(E + off0, CHUNK)], didx4.at[0])
    stage_idx(1, 1)
    issue_gathers(0, 0)

    def pair(t, carry):
        do_chunk(2 * t, 0, steady=True, guard_prev=True)
        do_chunk(2 * t + 1, 1, steady=True, guard_prev=False)
        return carry

    lax.fori_loop(0, (NCHUNK - 1) // 2, pair, 0)

    jl = NCHUNK - 1
    do_chunk(jl, 0, steady=False, guard_prev=False)
    wait_scatters(jl, 0)

    plsc.subcore_barrier()

    pltpu.sync_copy(acc_sh.at[pl.ds(s * ROWS_PER_TILE, ROWS_PER_TILE)],
                    acc_out.at[c, pl.ds(s * ROWS_PER_TILE, ROWS_PER_TILE)])
    pltpu.sync_copy(dsum_sh.at[pl.ds(s * ROWS_PER_TILE, ROWS_PER_TILE)],
                    dsum_out.at[c, pl.ds(s * ROWS_PER_TILE, ROWS_PER_TILE)])


def _edge_phase(h, asrc, adst, eidx, m_arr):
    mesh = plsc.VectorSubcoreMesh(core_axis_name="c", subcore_axis_name="s",
                                  num_cores=2, num_subcores=16)
    k = pl.kernel(
        _edge_body,
        out_type=[
            jax.ShapeDtypeStruct((2, N_PAD, D), jnp.float32),
            jax.ShapeDtypeStruct((2, N_PAD), jnp.float32),
        ],
        mesh=mesh,
        compiler_params=pltpu.CompilerParams(needs_layout_passes=False),
        scratch_types=[
            pltpu.VMEM((4, CHUNK), jnp.int32),
            pltpu.VMEM((4, CHUNK), jnp.int32),
            pltpu.VMEM((CHUNK, D), jnp.float32),
            pltpu.VMEM((CHUNK, D), jnp.float32),
            pltpu.VMEM((CHUNK,), jnp.float32),
            pltpu.VMEM((CHUNK,), jnp.float32),
            pltpu.VMEM((CHUNK,), jnp.float32),
            pltpu.VMEM((CHUNK,), jnp.float32),
            pltpu.VMEM((CHUNK,), jnp.float32),
            pltpu.VMEM((CHUNK,), jnp.float32),
            pltpu.VMEM((16,), jnp.float32),
            pltpu.SemaphoreType.DMA,
            pltpu.SemaphoreType.DMA,
            pltpu.SemaphoreType.DMA,
            pltpu.SemaphoreType.DMA,
            pltpu.SemaphoreType.DMA,
            pltpu.SemaphoreType.DMA,
            pltpu.SemaphoreType.DMA,
            pltpu.SemaphoreType.DMA,
            pltpu.SemaphoreType.DMA,
            pltpu.SemaphoreType.DMA,
            pltpu.SemaphoreType.DMA,
            pltpu.SemaphoreType.DMA,
            pltpu.SemaphoreType.DMA,
            pltpu.SemaphoreType.DMA,
            pltpu.VMEM_SHARED((N_PAD, D), jnp.float32),
            pltpu.VMEM_SHARED((N_PAD,), jnp.float32),
        ],
    )
    return k(h, asrc, adst, eidx, m_arr)


def _final_body(a0_ref, a1_ref, d0_ref, d1_ref, b_ref, out_ref):
    t = a0_ref[...] + a1_ref[...]
    d = d0_ref[...] + d1_ref[...] + 1e-16
    r = t / d + b_ref[...]
    out_ref[...] = jnp.where(r > 0.0, r, jnp.exp(r) - 1.0)


def _finalize(a0, a1, d0, d1, bias2):
    grid = (10,)
    return pl.pallas_call(
        _final_body,
        grid=grid,
        in_specs=[
            pl.BlockSpec((1000, 128), lambda i: (i, 0)),
            pl.BlockSpec((1000, 128), lambda i: (i, 0)),
            pl.BlockSpec((1000, 1), lambda i: (i, 0)),
            pl.BlockSpec((1000, 1), lambda i: (i, 0)),
            pl.BlockSpec((1, 128), lambda i: (0, 0)),
        ],
        out_specs=pl.BlockSpec((1000, 128), lambda i: (i, 0)),
        out_shape=jax.ShapeDtypeStruct((N, D), jnp.float32),
    )(a0, a1, d0, d1, bias2)


@jax.jit
def kernel(x, edge_index, W, a_src, a_dst, bias):
    a2 = jnp.zeros((D, 8), jnp.float32).at[:, 0].set(a_src).at[:, 1].set(a_dst)
    h, al, mx = _project(x, W, a2)
    asrc = al[:, 0]
    adst = al[:, 1]
    m = mx[0, 0] + mx[0, 1]
    m = jnp.where(m >= 0.0, m, 0.2 * m)
    m_arr = jnp.full((16,), m, jnp.float32)

    acc, dsum = _edge_phase(h, asrc, adst, edge_index.reshape(2 * E), m_arr)

    out = _finalize(acc[0], acc[1], dsum[0][:, None], dsum[1][:, None],
                    bias[None, :])
    return out


# speedup vs baseline: 44.7600x; 1.0033x over previous
import functools

import jax
import jax.numpy as jnp
from jax import lax
from jax.experimental import pallas as pl
from jax.experimental.pallas import tpu as pltpu
from jax.experimental.pallas import tpu_sc as plsc

N = 10000
E = 320000
D = 128
N_PAD = 10240
NW = 32
EDGES_PER_W = E // NW
CHUNK = 80
NCHUNK = EDGES_PER_W // CHUNK
ROWS_PER_TILE = N_PAD // 16


def _proj_body(x_ref, w_ref, a2_ref, h_ref, al_ref, mx_ref):
    i = pl.program_id(0)
    h = jnp.dot(x_ref[...], w_ref[...], preferred_element_type=jnp.float32)
    h_ref[...] = h
    al = jnp.dot(h, a2_ref[...], preferred_element_type=jnp.float32)
    al_ref[...] = al
    bm = jnp.max(al, axis=0, keepdims=True)

    @pl.when(i == 0)
    def _():
        mx_ref[...] = bm

    @pl.when(i > 0)
    def _():
        mx_ref[...] = jnp.maximum(mx_ref[...], bm)


def _project(x, W, a2):
    grid = (10,)
    return pl.pallas_call(
        _proj_body,
        grid=grid,
        in_specs=[
            pl.BlockSpec((1000, 128), lambda i: (i, 0)),
            pl.BlockSpec((128, 128), lambda i: (0, 0)),
            pl.BlockSpec((128, 8), lambda i: (0, 0)),
        ],
        out_specs=[
            pl.BlockSpec((1000, 128), lambda i: (i, 0)),
            pl.BlockSpec((1000, 8), lambda i: (i, 0)),
            pl.BlockSpec((1, 8), lambda i: (0, 0)),
        ],
        out_shape=[
            jax.ShapeDtypeStruct((N, D), jnp.float32),
            jax.ShapeDtypeStruct((N, 8), jnp.float32),
            jax.ShapeDtypeStruct((1, 8), jnp.float32),
        ],
    )(x, W, a2)


def _edge_body(h_hbm, asrc_hbm, adst_hbm, sidx_hbm, m_hbm,
               acc_out, dsum_out,
               sidx4, didx4, rows0, rows1, gbf0, gbf1, wb0, wb1, a1b0, a1b1,
               a2b0, a2b1, m_v,
               g0, g1, ga0, ga1, gb0, gb1, sa0, sa1, sd0, sd1,
               i0, i1, jj0, jj1,
               acc_sh, dsum_sh):
    c = lax.axis_index("c")
    s = lax.axis_index("s")
    wid = c * 16 + s

    rows = (rows0, rows1)
    gbf = (gbf0, gbf1)
    wb = (wb0, wb1)
    a1b = (a1b0, a1b1)
    a2b = (a2b0, a2b1)
    gsem = (g0, g1)
    gasem = (ga0, ga1)
    gbsem = (gb0, gb1)
    sasem = (sa0, sa1)
    sdsem = (sd0, sd1)
    isem = (i0, i1)
    jsem = (jj0, jj1)

    zero16 = jnp.zeros((16,), jnp.float32)

    def zrow(r, carry):
        for g in range(8):
            rows0[r, pl.ds(g * 16, 16)] = zero16
        return carry

    lax.fori_loop(0, 16, zrow, 0)
    z16 = rows0.at[pl.ds(0, 16)]

    def zcp(k, carry):
        pltpu.sync_copy(z16, acc_sh.at[pl.ds(s * ROWS_PER_TILE + k * 16, 16)])
        return carry

    lax.fori_loop(0, ROWS_PER_TILE // 16, zcp, 0)

    def zdcp(k, carry):
        pltpu.sync_copy(rows0.at[0],
                        dsum_sh.at[pl.ds(s * ROWS_PER_TILE + k * 128, 128)])
        return carry

    lax.fori_loop(0, ROWS_PER_TILE // 128, zdcp, 0)

    pltpu.sync_copy(m_hbm, m_v)
    mvec = m_v[...]

    plsc.subcore_barrier()

    def stage_idx(j, b):
        off = (wid * NCHUNK + j) * CHUNK
        pltpu.async_copy(sidx_hbm.at[pl.ds(off, CHUNK)], sidx4.at[j % 4],
                         isem[b])
        pltpu.async_copy(sidx_hbm.at[pl.ds(E + off, CHUNK)], didx4.at[j % 4],
                         jsem[b])

    def wait_idx(j, b):
        off = (wid * NCHUNK + j) * CHUNK
        pltpu.make_async_copy(sidx_hbm.at[pl.ds(off, CHUNK)],
                              sidx4.at[j % 4], isem[b]).wait()
        pltpu.make_async_copy(sidx_hbm.at[pl.ds(E + off, CHUNK)],
                              didx4.at[j % 4], jsem[b]).wait()

    def issue_gathers(j, b):
        pltpu.async_copy(h_hbm.at[sidx4.at[j % 4]], gbf[b], gsem[b])
        pltpu.async_copy(asrc_hbm.at[sidx4.at[j % 4]], a1b[b], gasem[b])
        pltpu.async_copy(adst_hbm.at[didx4.at[j % 4]], a2b[b], gbsem[b])

    def wait_acc_scatter(j, b):
        pltpu.make_async_copy(rows[b], acc_sh.at[didx4.at[j % 4]],
                              sasem[b]).wait()

    def wait_dsum_scatter(j, b):
        pltpu.make_async_copy(wb[b], dsum_sh.at[didx4.at[j % 4]],
                              sdsem[b]).wait()

    def do_chunk(j, b, steady):
        @pl.when(j > 1)
        def _():
            wait_dsum_scatter(j - 2, b)

        pltpu.make_async_copy(asrc_hbm.at[sidx4.at[j % 4]], a1b[b],
                              gasem[b]).wait()
        pltpu.make_async_copy(adst_hbm.at[didx4.at[j % 4]], a2b[b],
                              gbsem[b]).wait()
        ws = []
        for g in range(5):
            sl = pl.ds(g * 16, 16)
            e = a1b[b][sl] + a2b[b][sl]
            e = jnp.where(e >= 0.0, e, 0.2 * e)
            w = jnp.exp(e - mvec)
            wb[b][sl] = w
            ws.append(w)
        pltpu.async_copy(wb[b], dsum_sh.at[didx4.at[j % 4]], sdsem[b],
                         add=True)
        pltpu.make_async_copy(h_hbm.at[sidx4.at[j % 4]], gbf[b],
                              gsem[b]).wait()
        if steady:
            wait_idx(j + 1, 1 - b)
            issue_gathers(j + 1, 1 - b)

        @pl.when(j > 1)
        def _():
            wait_acc_scatter(j - 2, b)

        if steady:
            @pl.when(j + 2 < NCHUNK)
            def _():
                stage_idx(j + 2, b)

        for g in range(5):
            for l in range(16):
                wl = ws[g][l]
                r = g * 16 + l
                for f in range(8):
                    sl = pl.ds(f * 16, 16)
                    rows[b][r, sl] = gbf[b][r, sl] * wl
        pltpu.async_copy(rows[b], acc_sh.at[didx4.at[j % 4]], sasem[b],
                         add=True)

    off0 = wid * NCHUNK * CHUNK
    pltpu.sync_copy(sidx_hbm.at[pl.ds(off0, CHUNK)], sidx4.at[0])
    pltpu.sync_copy(sidx_hbm.at[pl.ds(E + off0, CHUNK)], didx4.at[0])
    stage_idx(1, 1)
    issue_gathers(0, 0)

    def pair(t, carry):
        do_chunk(2 * t, 0, steady=True)
        do_chunk(2 * t + 1, 1, steady=True)
        return carry

    lax.fori_loop(0, (NCHUNK - 1) // 2, pair, 0)

    jl = NCHUNK - 1
    do_chunk(jl, 0, steady=False)
    wait_dsum_scatter(jl - 1, 1)
    wait_acc_scatter(jl - 1, 1)
    wait_dsum_scatter(jl, 0)
    wait_acc_scatter(jl, 0)

    plsc.subcore_barrier()

    pltpu.sync_copy(acc_sh.at[pl.ds(s * ROWS_PER_TILE, ROWS_PER_TILE)],
                    acc_out.at[c, pl.ds(s * ROWS_PER_TILE, ROWS_PER_TILE)])
    pltpu.sync_copy(dsum_sh.at[pl.ds(s * ROWS_PER_TILE, ROWS_PER_TILE)],
                    dsum_out.at[c, pl.ds(s * ROWS_PER_TILE, ROWS_PER_TILE)])


def _edge_phase(h, asrc, adst, eidx, m_arr):
    mesh = plsc.VectorSubcoreMesh(core_axis_name="c", subcore_axis_name="s",
                                  num_cores=2, num_subcores=16)
    k = pl.kernel(
        _edge_body,
        out_type=[
            jax.ShapeDtypeStruct((2, N_PAD, D), jnp.float32),
            jax.ShapeDtypeStruct((2, N_PAD), jnp.float32),
        ],
        mesh=mesh,
        compiler_params=pltpu.CompilerParams(needs_layout_passes=False),
        scratch_types=[
            pltpu.VMEM((4, CHUNK), jnp.int32),
            pltpu.VMEM((4, CHUNK), jnp.int32),
            pltpu.VMEM((CHUNK, D), jnp.float32),
            pltpu.VMEM((CHUNK, D), jnp.float32),
            pltpu.VMEM((CHUNK, D), jnp.float32),
            pltpu.VMEM((CHUNK, D), jnp.float32),
            pltpu.VMEM((CHUNK,), jnp.float32),
            pltpu.VMEM((CHUNK,), jnp.float32),
            pltpu.VMEM((CHUNK,), jnp.float32),
            pltpu.VMEM((CHUNK,), jnp.float32),
            pltpu.VMEM((CHUNK,), jnp.float32),
            pltpu.VMEM((CHUNK,), jnp.float32),
            pltpu.VMEM((16,), jnp.float32),
            pltpu.SemaphoreType.DMA,
            pltpu.SemaphoreType.DMA,
            pltpu.SemaphoreType.DMA,
            pltpu.SemaphoreType.DMA,
            pltpu.SemaphoreType.DMA,
            pltpu.SemaphoreType.DMA,
            pltpu.SemaphoreType.DMA,
            pltpu.SemaphoreType.DMA,
            pltpu.SemaphoreType.DMA,
            pltpu.SemaphoreType.DMA,
            pltpu.SemaphoreType.DMA,
            pltpu.SemaphoreType.DMA,
            pltpu.SemaphoreType.DMA,
            pltpu.SemaphoreType.DMA,
            pltpu.VMEM_SHARED((N_PAD, D), jnp.float32),
            pltpu.VMEM_SHARED((N_PAD,), jnp.float32),
        ],
    )
    return k(h, asrc, adst, eidx, m_arr)


def _final_body(a0_ref, a1_ref, d0_ref, d1_ref, b_ref, out_ref):
    t = a0_ref[...] + a1_ref[...]
    d = d0_ref[...] + d1_ref[...] + 1e-16
    r = t / d + b_ref[...]
    out_ref[...] = jnp.where(r > 0.0, r, jnp.exp(r) - 1.0)


def _finalize(a0, a1, d0, d1, bias2):
    grid = (10,)
    return pl.pallas_call(
        _final_body,
        grid=grid,
        in_specs=[
            pl.BlockSpec((1000, 128), lambda i: (i, 0)),
            pl.BlockSpec((1000, 128), lambda i: (i, 0)),
            pl.BlockSpec((1000, 1), lambda i: (i, 0)),
            pl.BlockSpec((1000, 1), lambda i: (i, 0)),
            pl.BlockSpec((1, 128), lambda i: (0, 0)),
        ],
        out_specs=pl.BlockSpec((1000, 128), lambda i: (i, 0)),
        out_shape=jax.ShapeDtypeStruct((N, D), jnp.float32),
    )(a0, a1, d0, d1, bias2)


@jax.jit
def kernel(x, edge_index, W, a_src, a_dst, bias):
    a2 = jnp.zeros((D, 8), jnp.float32).at[:, 0].set(a_src).at[:, 1].set(a_dst)
    h, al, mx = _project(x, W, a2)
    asrc = al[:, 0]
    adst = al[:, 1]
    m = mx[0, 0] + mx[0, 1]
    m = jnp.where(m >= 0.0, m, 0.2 * m)
    m_arr = jnp.full((16,), m, jnp.float32)

    acc, dsum = _edge_phase(h, asrc, adst, edge_index.reshape(2 * E), m_arr)

    out = _finalize(acc[0], acc[1], dsum[0][:, None], dsum[1][:, None],
                    bias[None, :])
    return out


# speedup vs baseline: 49.5367x; 1.1067x over previous
import jax
import jax.numpy as jnp
from jax import lax
from jax.experimental import pallas as pl
from jax.experimental.pallas import tpu as pltpu
from jax.experimental.pallas import tpu_sc as plsc

N = 10000
E = 320000
D = 128
N_PAD = 10240
NW = 32
EDGES_PER_W = E // NW
CHUNK = 80
NCHUNK = EDGES_PER_W // CHUNK
ROWS_PER_TILE = N_PAD // 16


def _proj_body(x_ref, w_ref, a2_ref, h_ref, al_ref, mx_ref):
    i = pl.program_id(0)
    h = jnp.dot(x_ref[...], w_ref[...], preferred_element_type=jnp.float32)
    h_ref[...] = h
    al = jnp.dot(h, a2_ref[...], preferred_element_type=jnp.float32)
    al_ref[...] = al
    bm = jnp.max(al, axis=0, keepdims=True)

    @pl.when(i == 0)
    def _():
        mx_ref[...] = bm

    @pl.when(i > 0)
    def _():
        mx_ref[...] = jnp.maximum(mx_ref[...], bm)


def _project(x, W, a2):
    grid = (10,)
    return pl.pallas_call(
        _proj_body,
        grid=grid,
        in_specs=[
            pl.BlockSpec((1000, 128), lambda i: (i, 0)),
            pl.BlockSpec((128, 128), lambda i: (0, 0)),
            pl.BlockSpec((128, 8), lambda i: (0, 0)),
        ],
        out_specs=[
            pl.BlockSpec((1000, 128), lambda i: (i, 0)),
            pl.BlockSpec((1000, 8), lambda i: (i, 0)),
            pl.BlockSpec((1, 8), lambda i: (0, 0)),
        ],
        out_shape=[
            jax.ShapeDtypeStruct((N, D), jnp.float32),
            jax.ShapeDtypeStruct((N, 8), jnp.float32),
            jax.ShapeDtypeStruct((1, 8), jnp.float32),
        ],
    )(x, W, a2)


def _edge_body(h_hbm, asrc_hbm, adst_hbm, sidx_hbm, m_hbm,
               acc_out, dsum_out,
               sidx4, didx4, rows0, rows1, gbf0, gbf1, wb0, wb1, a1b0, a1b1,
               a2b0, a2b1, m_v,
               g0, g1, ga0, ga1, gb0, gb1, sa0, sa1, sd0, sd1,
               i0, i1, jj0, jj1,
               acc_sh, dsum_sh):
    c = lax.axis_index("c")
    s = lax.axis_index("s")
    wid = c * 16 + s

    rows = (rows0, rows1)
    gbf = (gbf0, gbf1)
    wb = (wb0, wb1)
    a1b = (a1b0, a1b1)
    a2b = (a2b0, a2b1)
    gsem = (g0, g1)
    gasem = (ga0, ga1)
    gbsem = (gb0, gb1)
    sasem = (sa0, sa1)
    sdsem = (sd0, sd1)
    isem = (i0, i1)
    jsem = (jj0, jj1)

    zero16 = jnp.zeros((16,), jnp.float32)

    def stage_idx(j, b):
        off = (wid * NCHUNK + j) * CHUNK
        pltpu.async_copy(sidx_hbm.at[pl.ds(off, CHUNK)], sidx4.at[j % 4],
                         isem[b])
        pltpu.async_copy(sidx_hbm.at[pl.ds(E + off, CHUNK)], didx4.at[j % 4],
                         jsem[b])

    def wait_idx(j, b):
        off = (wid * NCHUNK + j) * CHUNK
        pltpu.make_async_copy(sidx_hbm.at[pl.ds(off, CHUNK)],
                              sidx4.at[j % 4], isem[b]).wait()
        pltpu.make_async_copy(sidx_hbm.at[pl.ds(E + off, CHUNK)],
                              didx4.at[j % 4], jsem[b]).wait()

    def issue_gathers(j, b):
        pltpu.async_copy(h_hbm.at[sidx4.at[j % 4]], gbf[b], gsem[b])
        pltpu.async_copy(asrc_hbm.at[sidx4.at[j % 4]], a1b[b], gasem[b])
        pltpu.async_copy(adst_hbm.at[didx4.at[j % 4]], a2b[b], gbsem[b])

    def wait_acc_scatter(j, b):
        pltpu.make_async_copy(rows[b], acc_sh.at[didx4.at[j % 4]],
                              sasem[b]).wait()

    def wait_dsum_scatter(j, b):
        pltpu.make_async_copy(wb[b], dsum_sh.at[didx4.at[j % 4]],
                              sdsem[b]).wait()

    off0 = wid * NCHUNK * CHUNK
    pltpu.sync_copy(sidx_hbm.at[pl.ds(off0, CHUNK)], sidx4.at[0])
    pltpu.sync_copy(sidx_hbm.at[pl.ds(E + off0, CHUNK)], didx4.at[0])
    stage_idx(1, 1)
    issue_gathers(0, 0)
    pltpu.sync_copy(m_hbm, m_v)

    def zrow(r, carry):
        for g in range(8):
            rows0[r, pl.ds(g * 16, 16)] = zero16
        return carry

    lax.fori_loop(0, 16, zrow, 0)
    z16 = rows0.at[pl.ds(0, 16)]

    def zcp(k, carry):
        pltpu.sync_copy(z16, acc_sh.at[pl.ds(s * ROWS_PER_TILE + k * 16, 16)])
        return carry

    lax.fori_loop(0, ROWS_PER_TILE // 16, zcp, 0)

    def zdcp(k, carry):
        pltpu.sync_copy(rows0.at[0],
                        dsum_sh.at[pl.ds(s * ROWS_PER_TILE + k * 128, 128)])
        return carry

    lax.fori_loop(0, ROWS_PER_TILE // 128, zdcp, 0)

    mvec = m_v[...]

    plsc.subcore_barrier()

    def do_chunk(j, b, steady):
        @pl.when(j > 1)
        def _():
            wait_dsum_scatter(j - 2, b)

        pltpu.make_async_copy(asrc_hbm.at[sidx4.at[j % 4]], a1b[b],
                              gasem[b]).wait()
        pltpu.make_async_copy(adst_hbm.at[didx4.at[j % 4]], a2b[b],
                              gbsem[b]).wait()
        ws = []
        for g in range(5):
            sl = pl.ds(g * 16, 16)
            e = a1b[b][sl] + a2b[b][sl]
            e = jnp.where(e >= 0.0, e, 0.2 * e)
            w = jnp.exp(e - mvec)
            wb[b][sl] = w
            ws.append(w)
        pltpu.async_copy(wb[b], dsum_sh.at[didx4.at[j % 4]], sdsem[b],
                         add=True)
        if steady:
            wait_idx(j + 1, 1 - b)
            issue_gathers(j + 1, 1 - b)
        pltpu.make_async_copy(h_hbm.at[sidx4.at[j % 4]], gbf[b],
                              gsem[b]).wait()

        @pl.when(j > 1)
        def _():
            wait_acc_scatter(j - 2, b)

        if steady:
            @pl.when(j + 2 < NCHUNK)
            def _():
                stage_idx(j + 2, b)

        for g in range(5):
            for l in range(16):
                wl = ws[g][l]
                r = g * 16 + l
                for f in range(8):
                    sl = pl.ds(f * 16, 16)
                    rows[b][r, sl] = gbf[b][r, sl] * wl
        pltpu.async_copy(rows[b], acc_sh.at[didx4.at[j % 4]], sasem[b],
                         add=True)

    def pair(t, carry):
        do_chunk(2 * t, 0, steady=True)
        do_chunk(2 * t + 1, 1, steady=True)
        return carry

    lax.fori_loop(0, (NCHUNK - 1) // 2, pair, 0)

    jl = NCHUNK - 1
    do_chunk(jl, 0, steady=False)
    wait_dsum_scatter(jl - 1, 1)
    wait_acc_scatter(jl - 1, 1)
    wait_dsum_scatter(jl, 0)
    wait_acc_scatter(jl, 0)

    plsc.subcore_barrier()

    pltpu.sync_copy(acc_sh.at[pl.ds(s * ROWS_PER_TILE, ROWS_PER_TILE)],
                    acc_out.at[c, pl.ds(s * ROWS_PER_TILE, ROWS_PER_TILE)])
    pltpu.sync_copy(dsum_sh.at[pl.ds(s * ROWS_PER_TILE, ROWS_PER_TILE)],
                    dsum_out.at[c, pl.ds(s * ROWS_PER_TILE, ROWS_PER_TILE)])


def _edge_phase(h, asrc, adst, eidx, m_arr):
    mesh = plsc.VectorSubcoreMesh(core_axis_name="c", subcore_axis_name="s",
                                  num_cores=2, num_subcores=16)
    k = pl.kernel(
        _edge_body,
        out_type=[
            jax.ShapeDtypeStruct((2, N_PAD, D), jnp.float32),
            jax.ShapeDtypeStruct((2, N_PAD), jnp.float32),
        ],
        mesh=mesh,
        compiler_params=pltpu.CompilerParams(needs_layout_passes=False),
        scratch_types=[
            pltpu.VMEM((4, CHUNK), jnp.int32),
            pltpu.VMEM((4, CHUNK), jnp.int32),
            pltpu.VMEM((CHUNK, D), jnp.float32),
            pltpu.VMEM((CHUNK, D), jnp.float32),
            pltpu.VMEM((CHUNK, D), jnp.float32),
            pltpu.VMEM((CHUNK, D), jnp.float32),
            pltpu.VMEM((CHUNK,), jnp.float32),
            pltpu.VMEM((CHUNK,), jnp.float32),
            pltpu.VMEM((CHUNK,), jnp.float32),
            pltpu.VMEM((CHUNK,), jnp.float32),
            pltpu.VMEM((CHUNK,), jnp.float32),
            pltpu.VMEM((CHUNK,), jnp.float32),
            pltpu.VMEM((16,), jnp.float32),
            pltpu.SemaphoreType.DMA,
            pltpu.SemaphoreType.DMA,
            pltpu.SemaphoreType.DMA,
            pltpu.SemaphoreType.DMA,
            pltpu.SemaphoreType.DMA,
            pltpu.SemaphoreType.DMA,
            pltpu.SemaphoreType.DMA,
            pltpu.SemaphoreType.DMA,
            pltpu.SemaphoreType.DMA,
            pltpu.SemaphoreType.DMA,
            pltpu.SemaphoreType.DMA,
            pltpu.SemaphoreType.DMA,
            pltpu.SemaphoreType.DMA,
            pltpu.SemaphoreType.DMA,
            pltpu.VMEM_SHARED((N_PAD, D), jnp.float32),
            pltpu.VMEM_SHARED((N_PAD,), jnp.float32),
        ],
    )
    return k(h, asrc, adst, eidx, m_arr)


def _final_body(a0_ref, a1_ref, d0_ref, d1_ref, b_ref, out_ref):
    t = a0_ref[...] + a1_ref[...]
    d = d0_ref[...] + d1_ref[...] + 1e-16
    r = t / d + b_ref[...]
    out_ref[...] = jnp.where(r > 0.0, r, jnp.exp(r) - 1.0)


def _finalize(a0, a1, d0, d1, bias2):
    grid = (10,)
    return pl.pallas_call(
        _final_body,
        grid=grid,
        in_specs=[
            pl.BlockSpec((1000, 128), lambda i: (i, 0)),
            pl.BlockSpec((1000, 128), lambda i: (i, 0)),
            pl.BlockSpec((1000, 1), lambda i: (i, 0)),
            pl.BlockSpec((1000, 1), lambda i: (i, 0)),
            pl.BlockSpec((1, 128), lambda i: (0, 0)),
        ],
        out_specs=pl.BlockSpec((1000, 128), lambda i: (i, 0)),
        out_shape=jax.ShapeDtypeStruct((N, D), jnp.float32),
    )(a0, a1, d0, d1, bias2)


@jax.jit
def kernel(x, edge_index, W, a_src, a_dst, bias):
    a2 = jnp.zeros((D, 8), jnp.float32).at[:, 0].set(a_src).at[:, 1].set(a_dst)
    h, al, mx = _project(x, W, a2)
    asrc = al[:, 0]
    adst = al[:, 1]
    m = mx[0, 0] + mx[0, 1]
    m = jnp.where(m >= 0.0, m, 0.2 * m)
    m_arr = jnp.full((16,), m, jnp.float32)

    acc, dsum = _edge_phase(h, asrc, adst, edge_index.reshape(2 * E), m_arr)

    out = _finalize(acc[0], acc[1], dsum[0][:, None], dsum[1][:, None],
                    bias[None, :])
    return out


# speedup vs baseline: 50.8015x; 1.0255x over previous
import jax
import jax.numpy as jnp
from jax import lax
from jax.experimental import pallas as pl
from jax.experimental.pallas import tpu as pltpu
from jax.experimental.pallas import tpu_sc as plsc

N = 10000
E = 320000
D = 128
N_PAD = 10240
NW = 32
EDGES_PER_W = E // NW
CHUNK = 80
NCHUNK = EDGES_PER_W // CHUNK
ROWS_PER_TILE = N_PAD // 16


def _proj_body(x_ref, w_ref, a2_ref, h_ref, al_ref, mx_ref):
    i = pl.program_id(0)
    h = jnp.dot(x_ref[...], w_ref[...], preferred_element_type=jnp.float32)
    h_ref[...] = h
    al = jnp.dot(h, a2_ref[...], preferred_element_type=jnp.float32)
    al_ref[...] = al
    bm = jnp.max(al, axis=0, keepdims=True)

    @pl.when(i == 0)
    def _():
        mx_ref[...] = bm

    @pl.when(i > 0)
    def _():
        mx_ref[...] = jnp.maximum(mx_ref[...], bm)


def _project(x, W, a2):
    grid = (10,)
    return pl.pallas_call(
        _proj_body,
        grid=grid,
        in_specs=[
            pl.BlockSpec((1000, 128), lambda i: (i, 0)),
            pl.BlockSpec((128, 128), lambda i: (0, 0)),
            pl.BlockSpec((128, 8), lambda i: (0, 0)),
        ],
        out_specs=[
            pl.BlockSpec((1000, 128), lambda i: (i, 0)),
            pl.BlockSpec((1000, 8), lambda i: (i, 0)),
            pl.BlockSpec((1, 8), lambda i: (0, 0)),
        ],
        out_shape=[
            jax.ShapeDtypeStruct((N, D), jnp.float32),
            jax.ShapeDtypeStruct((N, 8), jnp.float32),
            jax.ShapeDtypeStruct((1, 8), jnp.float32),
        ],
    )(x, W, a2)


def _edge_body(h_hbm, asrc_hbm, adst_hbm, sidx_hbm, m_hbm,
               acc_out, dsum_out,
               sidx4, didx4, rows0, rows1, gbf0, gbf1, wb0, wb1, a1b0, a1b1,
               a2b0, a2b1, m_v,
               g0, g1, ga0, ga1, gb0, gb1, sa0, sa1, sd0, sd1,
               i0, i1, jj0, jj1,
               acc_sh, dsum_sh):
    c = lax.axis_index("c")
    s = lax.axis_index("s")
    wid = c * 16 + s

    rows = (rows0, rows1)
    gbf = (gbf0, gbf1)
    wb = (wb0, wb1)
    a1b = (a1b0, a1b1)
    a2b = (a2b0, a2b1)
    gsem = (g0, g1)
    gasem = (ga0, ga1)
    gbsem = (gb0, gb1)
    sasem = (sa0, sa1)
    sdsem = (sd0, sd1)
    isem = (i0, i1)
    jsem = (jj0, jj1)

    zero16 = jnp.zeros((16,), jnp.float32)

    def stage_idx(j, b):
        off = (wid * NCHUNK + j) * CHUNK
        pltpu.async_copy(sidx_hbm.at[pl.ds(off, CHUNK)], sidx4.at[j % 4],
                         isem[b])
        pltpu.async_copy(sidx_hbm.at[pl.ds(E + off, CHUNK)], didx4.at[j % 4],
                         jsem[b])

    def wait_idx(j, b):
        off = (wid * NCHUNK + j) * CHUNK
        pltpu.make_async_copy(sidx_hbm.at[pl.ds(off, CHUNK)],
                              sidx4.at[j % 4], isem[b]).wait()
        pltpu.make_async_copy(sidx_hbm.at[pl.ds(E + off, CHUNK)],
                              didx4.at[j % 4], jsem[b]).wait()

    def issue_gathers(j, b):
        pltpu.async_copy(h_hbm.at[sidx4.at[j % 4]], gbf[b], gsem[b])
        pltpu.async_copy(asrc_hbm.at[sidx4.at[j % 4]], a1b[b], gasem[b])
        pltpu.async_copy(adst_hbm.at[didx4.at[j % 4]], a2b[b], gbsem[b])

    def wait_acc_scatter(j, b):
        pltpu.make_async_copy(rows[b], acc_sh.at[didx4.at[j % 4]],
                              sasem[b]).wait()

    def wait_dsum_scatter(j, b):
        pltpu.make_async_copy(wb[b], dsum_sh.at[didx4.at[j % 4]],
                              sdsem[b]).wait()

    off0 = wid * NCHUNK * CHUNK
    pltpu.sync_copy(sidx_hbm.at[pl.ds(off0, CHUNK)], sidx4.at[0])
    pltpu.sync_copy(sidx_hbm.at[pl.ds(E + off0, CHUNK)], didx4.at[0])
    stage_idx(1, 1)
    issue_gathers(0, 0)
    pltpu.sync_copy(m_hbm, m_v)

    def zrow(r, carry):
        for g in range(8):
            rows0[r, pl.ds(g * 16, 16)] = zero16
        return carry

    lax.fori_loop(0, 16, zrow, 0)
    z16 = rows0.at[pl.ds(0, 16)]

    def zcp(k, carry):
        pltpu.sync_copy(z16, acc_sh.at[pl.ds(s * ROWS_PER_TILE + k * 16, 16)])
        return carry

    lax.fori_loop(0, ROWS_PER_TILE // 16, zcp, 0)

    def zdcp(k, carry):
        pltpu.sync_copy(rows0.at[0],
                        dsum_sh.at[pl.ds(s * ROWS_PER_TILE + k * 128, 128)])
        return carry

    lax.fori_loop(0, ROWS_PER_TILE // 128, zdcp, 0)

    mvec = m_v[...]

    plsc.subcore_barrier()

    def do_chunk(j, b, steady):
        if steady:
            wait_idx(j + 1, 1 - b)
            issue_gathers(j + 1, 1 - b)

        @pl.when(j > 1)
        def _():
            wait_dsum_scatter(j - 2, b)

        pltpu.make_async_copy(asrc_hbm.at[sidx4.at[j % 4]], a1b[b],
                              gasem[b]).wait()
        pltpu.make_async_copy(adst_hbm.at[didx4.at[j % 4]], a2b[b],
                              gbsem[b]).wait()
        ws = []
        for g in range(5):
            sl = pl.ds(g * 16, 16)
            e = a1b[b][sl] + a2b[b][sl]
            e = jnp.where(e >= 0.0, e, 0.2 * e)
            w = jnp.exp(e - mvec)
            wb[b][sl] = w
            ws.append(w)
        pltpu.async_copy(wb[b], dsum_sh.at[didx4.at[j % 4]], sdsem[b],
                         add=True)
        pltpu.make_async_copy(h_hbm.at[sidx4.at[j % 4]], gbf[b],
                              gsem[b]).wait()

        @pl.when(j > 1)
        def _():
            wait_acc_scatter(j - 2, b)

        if steady:
            @pl.when(j + 2 < NCHUNK)
            def _():
                stage_idx(j + 2, b)

        for g in range(5):
            for l in range(16):
                wl = ws[g][l]
                r = g * 16 + l
                for f in range(8):
                    sl = pl.ds(f * 16, 16)
                    rows[b][r, sl] = gbf[b][r, sl] * wl
        pltpu.async_copy(rows[b], acc_sh.at[didx4.at[j % 4]], sasem[b],
                         add=True)

    def pair(t, carry):
        do_chunk(2 * t, 0, steady=True)
        do_chunk(2 * t + 1, 1, steady=True)
        return carry

    lax.fori_loop(0, (NCHUNK - 1) // 2, pair, 0)

    jl = NCHUNK - 1
    do_chunk(jl, 0, steady=False)
    wait_dsum_scatter(jl - 1, 1)
    wait_acc_scatter(jl - 1, 1)
    wait_dsum_scatter(jl, 0)
    wait_acc_scatter(jl, 0)

    plsc.subcore_barrier()

    pltpu.sync_copy(acc_sh.at[pl.ds(s * ROWS_PER_TILE, ROWS_PER_TILE)],
                    acc_out.at[c, pl.ds(s * ROWS_PER_TILE, ROWS_PER_TILE)])
    pltpu.sync_copy(dsum_sh.at[pl.ds(s * ROWS_PER_TILE, ROWS_PER_TILE)],
                    dsum_out.at[c, pl.ds(s * ROWS_PER_TILE, ROWS_PER_TILE)])


def _edge_phase(h, asrc, adst, eidx, m_arr):
    mesh = plsc.VectorSubcoreMesh(core_axis_name="c", subcore_axis_name="s",
                                  num_cores=2, num_subcores=16)
    k = pl.kernel(
        _edge_body,
        out_type=[
            jax.ShapeDtypeStruct((2, N_PAD, D), jnp.float32),
            jax.ShapeDtypeStruct((2, N_PAD), jnp.float32),
        ],
        mesh=mesh,
        compiler_params=pltpu.CompilerParams(needs_layout_passes=False),
        scratch_types=[
            pltpu.VMEM((4, CHUNK), jnp.int32),
            pltpu.VMEM((4, CHUNK), jnp.int32),
            pltpu.VMEM((CHUNK, D), jnp.float32),
            pltpu.VMEM((CHUNK, D), jnp.float32),
            pltpu.VMEM((CHUNK, D), jnp.float32),
            pltpu.VMEM((CHUNK, D), jnp.float32),
            pltpu.VMEM((CHUNK,), jnp.float32),
            pltpu.VMEM((CHUNK,), jnp.float32),
            pltpu.VMEM((CHUNK,), jnp.float32),
            pltpu.VMEM((CHUNK,), jnp.float32),
            pltpu.VMEM((CHUNK,), jnp.float32),
            pltpu.VMEM((CHUNK,), jnp.float32),
            pltpu.VMEM((16,), jnp.float32),
            pltpu.SemaphoreType.DMA,
            pltpu.SemaphoreType.DMA,
            pltpu.SemaphoreType.DMA,
            pltpu.SemaphoreType.DMA,
            pltpu.SemaphoreType.DMA,
            pltpu.SemaphoreType.DMA,
            pltpu.SemaphoreType.DMA,
            pltpu.SemaphoreType.DMA,
            pltpu.SemaphoreType.DMA,
            pltpu.SemaphoreType.DMA,
            pltpu.SemaphoreType.DMA,
            pltpu.SemaphoreType.DMA,
            pltpu.SemaphoreType.DMA,
            pltpu.SemaphoreType.DMA,
            pltpu.VMEM_SHARED((N_PAD, D), jnp.float32),
            pltpu.VMEM_SHARED((N_PAD,), jnp.float32),
        ],
    )
    return k(h, asrc, adst, eidx, m_arr)


def _final_body(a0_ref, a1_ref, d0_ref, d1_ref, b_ref, out_ref):
    t = a0_ref[...] + a1_ref[...]
    d = d0_ref[...] + d1_ref[...] + 1e-16
    r = t / d + b_ref[...]
    out_ref[...] = jnp.where(r > 0.0, r, jnp.exp(r) - 1.0)


def _finalize(a0, a1, d0, d1, bias2):
    grid = (10,)
    return pl.pallas_call(
        _final_body,
        grid=grid,
        in_specs=[
            pl.BlockSpec((1000, 128), lambda i: (i, 0)),
            pl.BlockSpec((1000, 128), lambda i: (i, 0)),
            pl.BlockSpec((1000, 1), lambda i: (i, 0)),
            pl.BlockSpec((1000, 1), lambda i: (i, 0)),
            pl.BlockSpec((1, 128), lambda i: (0, 0)),
        ],
        out_specs=pl.BlockSpec((1000, 128), lambda i: (i, 0)),
        out_shape=jax.ShapeDtypeStruct((N, D), jnp.float32),
    )(a0, a1, d0, d1, bias2)


@jax.jit
def kernel(x, edge_index, W, a_src, a_dst, bias):
    a2 = jnp.zeros((D, 8), jnp.float32).at[:, 0].set(a_src).at[:, 1].set(a_dst)
    h, al, mx = _project(x, W, a2)
    asrc = al[:, 0]
    adst = al[:, 1]
    m = mx[0, 0] + mx[0, 1]
    m = jnp.where(m >= 0.0, m, 0.2 * m)
    m_arr = jnp.full((16,), m, jnp.float32)

    acc, dsum = _edge_phase(h, asrc, adst, edge_index.reshape(2 * E), m_arr)

    out = _finalize(acc[0], acc[1], dsum[0][:, None], dsum[1][:, None],
                    bias[None, :])
    return out
